# baseline (speedup 1.0000x reference)
import functools

import jax
import jax.numpy as jnp
from jax import lax
from jax.experimental import pallas as pl
from jax.experimental.pallas import tpu as pltpu
from jax.experimental.pallas import tpu_sc as plsc

_ROWS = 64
_COLS = 8192
_NC = 2
_NS = 16
_RPW = 2
_L = 16
_UNROLL = 8
_NACC = 4
_NSEG = 4
_SEG = _COLS // _NSEG
_SEG_GROUPS = _SEG // (_L * _UNROLL)

_INF = float("inf")


def _lex_better(ka, kb, bka, bkb):
    return (ka < bka) | ((ka == bka) & (kb < bkb))


def _combine(a, b):
    aka, akb, axl, axu = a
    bka, bkb, bxl, bxu = b
    better = _lex_better(bka, bkb, aka, akb)
    return (
        jnp.where(better, bka, aka),
        jnp.where(better, bkb, akb),
        jnp.where(better, bxl, axl),
        jnp.where(better, bxu, axu),
    )


def _fresh_accs():
    return tuple(
        (
            jnp.full((_L,), _INF, jnp.float32),
            jnp.full((_L,), _INF, jnp.float32),
            jnp.zeros((_L,), jnp.float32),
            jnp.zeros((_L,), jnp.float32),
        )
        for _ in range(_NACC)
    )


def _scan_segment(xlv, xuv, seg, accs):
    def body(g, carry):
        accs = list(carry)
        base = pl.multiple_of(seg * _SEG + g * (_L * _UNROLL), _L * _UNROLL)
        for j in range(_UNROLL):
            xlc = xlv[pl.ds(base + j * _L, _L)]
            xuc = xuv[pl.ds(base + j * _L, _L)]
            ka = jnp.float32(0.7) * xlc + jnp.float32(0.3) * xuc
            kb = (xlc + xuc) - ka
            bka, bkb, bxl, bxu = accs[j % _NACC]
            better = _lex_better(ka, kb, bka, bkb)
            accs[j % _NACC] = (
                jnp.where(better, ka, bka),
                jnp.where(better, kb, bkb),
                jnp.where(better, xlc, bxl),
                jnp.where(better, xuc, bxu),
            )
        return tuple(accs)

    return lax.fori_loop(0, _SEG_GROUPS, body, accs)


def _finish_row(accs, rot):
    accs = list(accs)
    while len(accs) > 1:
        accs = [_combine(accs[i], accs[i + 1]) for i in range(0, len(accs), 2)]
    bka, bkb, bxl, bxu = accs[0]
    for s in (1, 2, 4, 8):
        for i, v in enumerate((bka, bkb, bxl, bxu)):
            rot[pl.ds(32 * i, _L)] = v
            rot[pl.ds(32 * i + _L, _L)] = v
        w = (
            rot[pl.ds(s, _L)],
            rot[pl.ds(32 + s, _L)],
            rot[pl.ds(64 + s, _L)],
            rot[pl.ds(96 + s, _L)],
        )
        bka, bkb, bxl, bxu = _combine((bka, bkb, bxl, bxu), w)
    return bxl, bxu


def _sc_body(xl_hbm, xu_hbm, outl, outu, xl0, xu0, xl1, xu1, rot, resb, asm,
             shared, *sems):
    cc = lax.axis_index("c")
    ss = lax.axis_index("s")
    row0 = (cc * _NS + ss) * _RPW

    pairs = [(r, seg) for r in range(_RPW) for seg in range(_NSEG)]
    bufs = [(xl0, xu0), (xl1, xu1)]

    def fire(i):
        r, seg = pairs[i]
        xlb, xub = bufs[r]
        sl = pl.ds(seg * _SEG, _SEG)
        sem = sems[i % 4]
        return (
            pltpu.async_copy(xl_hbm.at[row0 + r, sl], xlb.at[sl], sem),
            pltpu.async_copy(xu_hbm.at[row0 + r, sl], xub.at[sl], sem),
        )

    inflight = {i: fire(i) for i in range(4)}

    lanes = lax.iota(jnp.int32, _L)
    winners = []
    accs = _fresh_accs()
    for i, (r, seg) in enumerate(pairs):
        cpa, cpb = inflight.pop(i)
        cpa.wait()
        cpb.wait()
        if i + 4 < len(pairs):
            inflight[i + 4] = fire(i + 4)
        xlb, xub = bufs[r]
        accs = _scan_segment(xlb, xub, seg, accs)
        if seg == _NSEG - 1:
            winners.append(_finish_row(accs, rot))
            accs = _fresh_accs()

    (xl_w0, xu_w0), (xl_w1, xu_w1) = winners

    acc = jnp.where(lanes == 0, xl_w0, jnp.zeros((_L,), jnp.float32))
    acc = jnp.where(lanes == 1, xl_w1, acc)
    acc = jnp.where(lanes == 2, xu_w0, acc)
    acc = jnp.where(lanes == 3, xu_w1, acc)
    resb[...] = acc
    pltpu.sync_copy(resb, shared.at[pl.ds(ss * _L, _L)])
    plsc.subcore_barrier()

    @pl.when(ss == 0)
    def _():
        pltpu.sync_copy(shared, asm.at[pl.ds(0, _NS * _L)])
        accl0 = jnp.zeros((_L,), jnp.float32)
        accl1 = jnp.zeros((_L,), jnp.float32)
        accu0 = jnp.zeros((_L,), jnp.float32)
        accu1 = jnp.zeros((_L,), jnp.float32)
        for r in range(_NS):
            vr = asm[pl.ds(r * _L, _L)]
            asm[pl.ds(256, _L)] = vr
            asm[pl.ds(256 + _L, _L)] = vr
            lane0 = (2 * r) % _L
            m = (lanes == lane0) | (lanes == lane0 + 1)
            rl = asm[pl.ds(256 + (_L - lane0) % _L, _L)]
            ru = asm[pl.ds(256 + (_L - lane0) % _L + 2, _L)]
            if r < 8:
                accl0 = jnp.where(m, rl, accl0)
                accu0 = jnp.where(m, ru, accu0)
            else:
                accl1 = jnp.where(m, rl, accl1)
                accu1 = jnp.where(m, ru, accu1)
        asm[pl.ds(0, _L)] = accl0
        asm[pl.ds(_L, _L)] = accl1
        asm[pl.ds(2 * _L, _L)] = accu0
        asm[pl.ds(3 * _L, _L)] = accu1
        base = cc * _NS * _RPW
        pltpu.sync_copy(asm.at[pl.ds(0, 2 * _L)], outl.at[pl.ds(base, 2 * _L)])
        pltpu.sync_copy(asm.at[pl.ds(2 * _L, 2 * _L)],
                        outu.at[pl.ds(base, 2 * _L)])


_sc_call = functools.partial(
    pl.kernel,
    mesh=plsc.VectorSubcoreMesh(core_axis_name="c", subcore_axis_name="s"),
    out_type=(
        jax.ShapeDtypeStruct((_ROWS,), jnp.float32),
        jax.ShapeDtypeStruct((_ROWS,), jnp.float32),
    ),
    scratch_types=[
        pltpu.VMEM((_COLS,), jnp.float32),
        pltpu.VMEM((_COLS,), jnp.float32),
        pltpu.VMEM((_COLS,), jnp.float32),
        pltpu.VMEM((_COLS,), jnp.float32),
        pltpu.VMEM((128,), jnp.float32),
        pltpu.VMEM((_L,), jnp.float32),
        pltpu.VMEM((256 + 2 * _L + 2,), jnp.float32),
        pltpu.VMEM_SHARED((_NS * _L,), jnp.float32),
        pltpu.SemaphoreType.DMA,
        pltpu.SemaphoreType.DMA,
        pltpu.SemaphoreType.DMA,
        pltpu.SemaphoreType.DMA,
    ],
)(_sc_body)


@jax.jit
def kernel(xl, xu):
    outl, outu = _sc_call(xl, xu)
    return outl.reshape(_ROWS, 1), outu.reshape(_ROWS, 1)


# speedup vs baseline: 3.4696x; 3.4696x over previous
import jax
import jax.numpy as jnp
from jax.experimental import pallas as pl
from jax.experimental.pallas import tpu as pltpu

_ROWS = 64
_COLS = 8192
_INF = float("inf")


def _tc_body(xl_ref, xu_ref, outl_ref, outu_ref):
    xl = xl_ref[...]
    xu = xu_ref[...]
    ka = jnp.float32(0.7) * xl + jnp.float32(0.3) * xu
    minka = jnp.min(ka, axis=1, keepdims=True)
    kb = jnp.float32(0.3) * xl + jnp.float32(0.7) * xu
    kbm = jnp.where(ka == minka, kb, _INF)
    minkb = jnp.min(kbm, axis=1, keepdims=True)
    sel = kbm == minkb
    outl_ref[...] = jnp.min(jnp.where(sel, xl, _INF), axis=1, keepdims=True)
    outu_ref[...] = jnp.min(jnp.where(sel, xu, _INF), axis=1, keepdims=True)


@jax.jit
def kernel(xl, xu):
    return pl.pallas_call(
        _tc_body,
        out_shape=(
            jax.ShapeDtypeStruct((_ROWS, 1), jnp.float32),
            jax.ShapeDtypeStruct((_ROWS, 1), jnp.float32),
        ),
    )(xl, xu)
